# baseline (speedup 1.0000x reference)
import functools

import jax
import jax.numpy as jnp
from jax import lax
from jax.experimental import pallas as pl
from jax.experimental.pallas import tpu as pltpu
from jax.experimental.pallas import tpu_sc as plsc

VOCAB = 100000
DIM = 128
BATCH = 4096
N_NEG = 20
LANES = 16
NC = 2
NS = 16
NW = NC * NS
BPW = BATCH // NW
CHUNKS = DIM // LANES
GROUPS = BPW // LANES


def _sc_scores(u_idx, v_idx, neg3, rowid, u_weight, v_weight):
    mesh = plsc.VectorSubcoreMesh(core_axis_name="c", subcore_axis_name="s")

    @functools.partial(
        pl.kernel,
        out_type=(
            jax.ShapeDtypeStruct((BATCH,), jnp.float32),
            jax.ShapeDtypeStruct((BATCH,), jnp.float32),
        ),
        mesh=mesh,
        compiler_params=pltpu.CompilerParams(needs_layout_passes=False),
        scratch_types=[
            pltpu.VMEM((BPW,), jnp.int32),
            pltpu.VMEM((BPW,), jnp.int32),
            pltpu.VMEM((N_NEG, BPW), jnp.int32),
            pltpu.VMEM((N_NEG, BPW), jnp.int32),
            pltpu.VMEM((BPW, DIM), jnp.float32),
            pltpu.VMEM((BPW, DIM), jnp.float32),
            pltpu.VMEM((BPW, DIM), jnp.float32),
            pltpu.VMEM_SHARED((NS * BPW, DIM), jnp.float32),
            pltpu.VMEM((BPW, DIM), jnp.float32),
            pltpu.VMEM((BPW, DIM), jnp.float32),
            pltpu.VMEM((BPW, DIM), jnp.float32),
            pltpu.VMEM((BPW,), jnp.float32),
            pltpu.VMEM((BPW,), jnp.float32),
            pltpu.SemaphoreType.DMA,
            pltpu.SemaphoreType.DMA,
            pltpu.SemaphoreType.DMA,
            pltpu.SemaphoreType.DMA,
            pltpu.SemaphoreType.DMA,
            pltpu.SemaphoreType.DMA,
            pltpu.SemaphoreType.DMA,
            pltpu.SemaphoreType.DMA,
        ],
    )
    def scores_kernel(u_idx_hbm, v_idx_hbm, neg3_hbm, rowid_hbm, u_w, v_w,
                      score_hbm, nscore_hbm,
                      uidx_v, vidx_v, nidx_v, idr_v, u_rows, v_rows, negsum,
                      negsh, rb0, rb1, rb2, score_v, nscore_v,
                      semu, semv,
                      gs0, gs1, gs2, ss0, ss1, ss2):
        sub = lax.axis_index("s")
        wid = sub * NC + lax.axis_index("c")
        base = pl.multiple_of(wid * BPW, BPW)
        shbase = pl.multiple_of(sub * BPW, BPW)

        bufs = (rb0, rb1, rb2)
        gsems = (gs0, gs1, gs2)
        ssems = (ss0, ss1, ss2)
        NRING = 3

        pltpu.sync_copy(u_idx_hbm.at[pl.ds(base, BPW)], uidx_v)
        pltpu.sync_copy(v_idx_hbm.at[pl.ds(base, BPW)], vidx_v)
        pltpu.sync_copy(neg3_hbm.at[wid], nidx_v)
        pltpu.sync_copy(rowid_hbm, idr_v)

        cu = pltpu.async_copy(u_w.at[uidx_v], u_rows, semu)
        cv = pltpu.async_copy(v_w.at[vidx_v], v_rows, semv)
        gpend = {}
        for n in range(NRING):
            gpend[n] = pltpu.async_copy(v_w.at[nidx_v.at[n]], bufs[n], gsems[n])

        lane_iota = jnp.arange(LANES, dtype=jnp.int32)

        def id_body(r, carry):
            for c in range(CHUNKS):
                sl = pl.ds(c * LANES, LANES)
                idr_v[r, sl] = idr_v[r, sl] + shbase
            return carry

        lax.fori_loop(0, N_NEG, id_body, 0)

        zero_chunk = jnp.zeros((LANES,), jnp.float32)

        def z_body(i, carry):
            for c in range(CHUNKS):
                negsum[i, pl.ds(c * LANES, LANES)] = zero_chunk
            return carry

        lax.fori_loop(0, BPW, z_body, 0)
        pltpu.sync_copy(negsum, negsh.at[pl.ds(shbase, BPW)])

        cu.wait()
        cv.wait()

        def pos_body(g, carry):
            sp = jnp.zeros((LANES,), jnp.float32)
            for i in range(LANES):
                item = g * LANES + i
                accp = jnp.zeros((LANES,), jnp.float32)
                for c in range(CHUNKS):
                    sl = pl.ds(c * LANES, LANES)
                    accp = accp + u_rows[item, sl] * v_rows[item, sl]
                sp = jnp.where(lane_iota == i, jnp.sum(accp), sp)
            score_v[pl.ds(g * LANES, LANES)] = sp
            return carry

        lax.fori_loop(0, GROUPS, pos_body, 0)
        pltpu.sync_copy(score_v, score_hbm.at[pl.ds(base, BPW)])

        spend = {}
        for n in range(N_NEG):
            b = n % NRING
            gpend[b].wait()
            spend[b] = pltpu.async_copy(
                bufs[b], negsh.at[idr_v.at[n]], ssems[b], add=True
            )
            nxt = n + NRING
            if nxt < N_NEG:
                spend[b].wait()
                del spend[b]
                gpend[b] = pltpu.async_copy(
                    v_w.at[nidx_v.at[nxt]], bufs[b], gsems[b]
                )
        for b in sorted(spend):
            spend[b].wait()
        pltpu.sync_copy(negsh.at[pl.ds(shbase, BPW)], negsum)

        def neg_body(g, carry):
            sn = jnp.zeros((LANES,), jnp.float32)
            for i in range(LANES):
                item = g * LANES + i
                accn = jnp.zeros((LANES,), jnp.float32)
                for c in range(CHUNKS):
                    sl = pl.ds(c * LANES, LANES)
                    accn = accn + u_rows[item, sl] * negsum[item, sl]
                sn = jnp.where(lane_iota == i, jnp.sum(accn), sn)
            nscore_v[pl.ds(g * LANES, LANES)] = sn
            return carry

        lax.fori_loop(0, GROUPS, neg_body, 0)
        pltpu.sync_copy(nscore_v, nscore_hbm.at[pl.ds(base, BPW)])

    return scores_kernel(u_idx, v_idx, neg3, rowid, u_weight, v_weight)


def _loss_tc_kernel(s_ref, n_ref, o_ref):
    s = s_ref[...]
    ns = n_ref[...]

    def logsig(x):
        return jnp.minimum(x, 0.0) - jnp.log1p(jnp.exp(-jnp.abs(x)))

    total = jnp.sum(logsig(s) + logsig(-ns))
    o_ref[...] = jnp.full((1, 1), -total / BATCH, jnp.float32)


def kernel(pos, v_neg, u_weight, v_weight):
    u_idx = pos[:, 0].astype(jnp.int32)
    v_idx = pos[:, 1].astype(jnp.int32)
    neg3 = v_neg.astype(jnp.int32).reshape(NW, N_NEG, BPW)
    rowid = (jnp.arange(N_NEG * BPW, dtype=jnp.int32) // N_NEG).reshape(
        N_NEG, BPW
    )
    score, nscore = _sc_scores(u_idx, v_idx, neg3, rowid, u_weight, v_weight)

    out = pl.pallas_call(
        _loss_tc_kernel,
        out_shape=jax.ShapeDtypeStruct((1, 1), jnp.float32),
    )(score.reshape(NW, BPW), nscore.reshape(NW, BPW))
    return out[0, 0]


# speedup vs baseline: 1.0882x; 1.0882x over previous
import functools

import jax
import jax.numpy as jnp
from jax import lax
from jax.experimental import pallas as pl
from jax.experimental.pallas import tpu as pltpu
from jax.experimental.pallas import tpu_sc as plsc

VOCAB = 100000
DIM = 128
BATCH = 4096
N_NEG = 20
LANES = 16
NC = 2
NS = 16
NW = NC * NS
BPW = BATCH // NW
CHUNKS = DIM // LANES
GROUPS = BPW // LANES


def _sc_scores(posr, negr, u_weight, v_weight):
    mesh = plsc.VectorSubcoreMesh(core_axis_name="c", subcore_axis_name="s")

    @functools.partial(
        pl.kernel,
        out_type=(
            jax.ShapeDtypeStruct((BATCH,), jnp.float32),
            jax.ShapeDtypeStruct((BATCH,), jnp.float32),
        ),
        mesh=mesh,
        compiler_params=pltpu.CompilerParams(needs_layout_passes=False),
        scratch_types=[
            pltpu.VMEM((BPW,), jnp.int32),
            pltpu.VMEM((BPW,), jnp.int32),
            pltpu.VMEM((2 * BPW,), jnp.int32),
            pltpu.VMEM((N_NEG * BPW,), jnp.int32),
            pltpu.VMEM((N_NEG, BPW), jnp.int32),
            pltpu.VMEM((1, BPW), jnp.int32),
            pltpu.VMEM((BPW, DIM), jnp.float32),
            pltpu.VMEM((BPW, DIM), jnp.float32),
            pltpu.VMEM((BPW, DIM), jnp.float32),
            pltpu.VMEM_SHARED((NS * BPW, DIM), jnp.float32),
            pltpu.VMEM((BPW, DIM), jnp.float32),
            pltpu.VMEM((BPW, DIM), jnp.float32),
            pltpu.VMEM((BPW, DIM), jnp.float32),
            pltpu.VMEM((BPW,), jnp.float32),
            pltpu.VMEM((BPW,), jnp.float32),
            pltpu.SemaphoreType.DMA,
            pltpu.SemaphoreType.DMA,
            pltpu.SemaphoreType.DMA,
            pltpu.SemaphoreType.DMA,
            pltpu.SemaphoreType.DMA,
            pltpu.SemaphoreType.DMA,
            pltpu.SemaphoreType.DMA,
            pltpu.SemaphoreType.DMA,
            pltpu.SemaphoreType.DMA,
        ],
    )
    def scores_kernel(pos_hbm, neg_hbm, u_w, v_w,
                      score_hbm, nscore_hbm,
                      uidx_v, vidx_v, posflat_v, rawn_v, nidx_v, ident_v,
                      u_rows, v_rows, negsum,
                      negsh, rb0, rb1, rb2, score_v, nscore_v,
                      semu, semv, sem0,
                      gs0, gs1, gs2, ss0, ss1, ss2):
        sub = lax.axis_index("s")
        wid = sub * NC + lax.axis_index("c")
        base = pl.multiple_of(wid * BPW, BPW)
        shbase = pl.multiple_of(sub * BPW, BPW)

        bufs = (rb0, rb1, rb2)
        gsems = (gs0, gs1, gs2)
        ssems = (ss0, ss1, ss2)
        NRING = 3

        lane_iota = jnp.arange(LANES, dtype=jnp.int32)

        pltpu.sync_copy(pos_hbm.at[wid], posflat_v)
        pltpu.sync_copy(neg_hbm.at[wid], rawn_v)

        for c in range(CHUNKS):
            sl = pl.ds(c * LANES, LANES)
            j2 = 2 * (c * LANES + lane_iota)
            uidx_v[sl] = plsc.load_gather(posflat_v, [j2])
            vidx_v[sl] = plsc.load_gather(posflat_v, [j2 + 1])

        def tr_round(r):
            for c in range(CHUNKS):
                sl = pl.ds(c * LANES, LANES)
                jn = (c * LANES + lane_iota) * N_NEG + r
                nidx_v[r, sl] = plsc.load_gather(rawn_v, [jn])

        for r in range(NRING + 1):
            tr_round(r)

        cu = pltpu.async_copy(u_w.at[uidx_v], u_rows, semu)
        cv = pltpu.async_copy(v_w.at[vidx_v], v_rows, semv)
        c0 = pltpu.async_copy(v_w.at[nidx_v.at[0]], negsum, sem0)
        gpend = {}
        for n in range(1, 1 + NRING):
            b = (n - 1) % NRING
            gpend[b] = pltpu.async_copy(v_w.at[nidx_v.at[n]], bufs[b], gsems[b])

        for r in range(NRING + 1, N_NEG):
            tr_round(r)
        for g in range(GROUPS):
            ident_v[0, pl.ds(g * LANES, LANES)] = shbase + g * LANES + lane_iota

        cu.wait()
        cv.wait()

        def pos_body(g, carry):
            sp = jnp.zeros((LANES,), jnp.float32)
            for i in range(LANES):
                item = g * LANES + i
                accp = jnp.zeros((LANES,), jnp.float32)
                for c in range(CHUNKS):
                    sl = pl.ds(c * LANES, LANES)
                    accp = accp + u_rows[item, sl] * v_rows[item, sl]
                sp = jnp.where(lane_iota == i, jnp.sum(accp), sp)
            score_v[pl.ds(g * LANES, LANES)] = sp
            return carry

        lax.fori_loop(0, GROUPS, pos_body, 0)
        pltpu.sync_copy(score_v, score_hbm.at[pl.ds(base, BPW)])

        c0.wait()
        pltpu.sync_copy(negsum, negsh.at[pl.ds(shbase, BPW)])
        spend = {}
        for n in range(1, N_NEG):
            b = (n - 1) % NRING
            gpend[b].wait()
            spend[b] = pltpu.async_copy(
                bufs[b], negsh.at[ident_v.at[0]], ssems[b], add=True
            )
            nxt = n + NRING
            if nxt < N_NEG:
                spend[b].wait()
                del spend[b]
                gpend[b] = pltpu.async_copy(
                    v_w.at[nidx_v.at[nxt]], bufs[b], gsems[b]
                )
        for b in sorted(spend):
            spend[b].wait()
        pltpu.sync_copy(negsh.at[pl.ds(shbase, BPW)], negsum)

        def neg_body(g, carry):
            sn = jnp.zeros((LANES,), jnp.float32)
            for i in range(LANES):
                item = g * LANES + i
                accn = jnp.zeros((LANES,), jnp.float32)
                for c in range(CHUNKS):
                    sl = pl.ds(c * LANES, LANES)
                    accn = accn + u_rows[item, sl] * negsum[item, sl]
                sn = jnp.where(lane_iota == i, jnp.sum(accn), sn)
            nscore_v[pl.ds(g * LANES, LANES)] = sn
            return carry

        lax.fori_loop(0, GROUPS, neg_body, 0)
        pltpu.sync_copy(nscore_v, nscore_hbm.at[pl.ds(base, BPW)])

    return scores_kernel(posr, negr, u_weight, v_weight)


def _loss_tc_kernel(s_ref, n_ref, o_ref):
    s = s_ref[...]
    ns = n_ref[...]

    def logsig(x):
        return jnp.minimum(x, 0.0) - jnp.log1p(jnp.exp(-jnp.abs(x)))

    total = jnp.sum(logsig(s) + logsig(-ns))
    o_ref[...] = jnp.full((1, 1), -total / BATCH, jnp.float32)


def kernel(pos, v_neg, u_weight, v_weight):
    posr = pos.astype(jnp.int32).reshape(NW, 2 * BPW)
    negr = v_neg.astype(jnp.int32).reshape(NW, N_NEG * BPW)
    score, nscore = _sc_scores(posr, negr, u_weight, v_weight)

    out = pl.pallas_call(
        _loss_tc_kernel,
        out_shape=jax.ShapeDtypeStruct((1, 1), jnp.float32),
    )(score.reshape(NW, BPW), nscore.reshape(NW, BPW))
    return out[0, 0]


# speedup vs baseline: 1.4232x; 1.3079x over previous
import functools

import jax
import jax.numpy as jnp
from jax import lax
from jax.experimental import pallas as pl
from jax.experimental.pallas import tpu as pltpu
from jax.experimental.pallas import tpu_sc as plsc

VOCAB = 100000
DIM = 128
BATCH = 4096
N_NEG = 20
LANES = 16
NC = 2
NS = 16
NW = NC * NS
BPW = BATCH // NW
CHUNKS = DIM // LANES
GROUPS = BPW // LANES
NSEM = 4


def _sc_scores(u_idx, v_idx, neg3, u_weight, v_weight):
    mesh = plsc.VectorSubcoreMesh(core_axis_name="c", subcore_axis_name="s")

    @functools.partial(
        pl.kernel,
        out_type=(
            jax.ShapeDtypeStruct((BATCH,), jnp.float32),
            jax.ShapeDtypeStruct((BATCH,), jnp.float32),
        ),
        mesh=mesh,
        compiler_params=pltpu.CompilerParams(needs_layout_passes=False),
        scratch_types=[
            pltpu.VMEM((BPW,), jnp.int32),
            pltpu.VMEM((BPW,), jnp.int32),
            pltpu.VMEM((N_NEG, BPW), jnp.int32),
            pltpu.VMEM((BPW, DIM), jnp.float32),
            pltpu.VMEM((BPW, DIM), jnp.float32),
            pltpu.VMEM((BPW, DIM), jnp.float32),
            pltpu.VMEM((BPW,), jnp.float32),
            pltpu.VMEM((BPW,), jnp.float32),
            pltpu.SemaphoreType.DMA,
            pltpu.SemaphoreType.DMA,
            pltpu.SemaphoreType.DMA,
            pltpu.SemaphoreType.DMA,
            pltpu.SemaphoreType.DMA,
            pltpu.SemaphoreType.DMA,
            pltpu.SemaphoreType.DMA,
        ],
    )
    def scores_kernel(u_idx_hbm, v_idx_hbm, neg3_hbm, u_w, v_w,
                      score_hbm, nscore_hbm,
                      uidx_v, vidx_v, nidx_v, u_rows, v_rows, negsum,
                      score_v, nscore_v,
                      semu, semv, sem0, as0, as1, as2, as3):
        sub = lax.axis_index("s")
        wid = sub * NC + lax.axis_index("c")
        base = pl.multiple_of(wid * BPW, BPW)
        asems = (as0, as1, as2, as3)

        pltpu.sync_copy(u_idx_hbm.at[pl.ds(base, BPW)], uidx_v)
        pltpu.sync_copy(v_idx_hbm.at[pl.ds(base, BPW)], vidx_v)
        pltpu.sync_copy(neg3_hbm.at[wid], nidx_v)

        cu = pltpu.async_copy(u_w.at[uidx_v], u_rows, semu)
        cv = pltpu.async_copy(v_w.at[vidx_v], v_rows, semv)
        c0 = pltpu.async_copy(v_w.at[nidx_v.at[0]], negsum, sem0)

        c0.wait()
        pend = []
        for n in range(1, N_NEG):
            pend.append(
                pltpu.async_copy(
                    v_w.at[nidx_v.at[n]], negsum, asems[(n - 1) % NSEM],
                    add=True,
                )
            )

        lane_iota = jnp.arange(LANES, dtype=jnp.int32)

        cu.wait()
        cv.wait()

        def pos_body(g, carry):
            sp = jnp.zeros((LANES,), jnp.float32)
            for i in range(LANES):
                item = g * LANES + i
                accp = jnp.zeros((LANES,), jnp.float32)
                for c in range(CHUNKS):
                    sl = pl.ds(c * LANES, LANES)
                    accp = accp + u_rows[item, sl] * v_rows[item, sl]
                sp = jnp.where(lane_iota == i, jnp.sum(accp), sp)
            score_v[pl.ds(g * LANES, LANES)] = sp
            return carry

        lax.fori_loop(0, GROUPS, pos_body, 0)
        pltpu.sync_copy(score_v, score_hbm.at[pl.ds(base, BPW)])

        for c in pend:
            c.wait()

        def neg_body(g, carry):
            sn = jnp.zeros((LANES,), jnp.float32)
            for i in range(LANES):
                item = g * LANES + i
                accn = jnp.zeros((LANES,), jnp.float32)
                for c in range(CHUNKS):
                    sl = pl.ds(c * LANES, LANES)
                    accn = accn + u_rows[item, sl] * negsum[item, sl]
                sn = jnp.where(lane_iota == i, jnp.sum(accn), sn)
            nscore_v[pl.ds(g * LANES, LANES)] = sn
            return carry

        lax.fori_loop(0, GROUPS, neg_body, 0)
        pltpu.sync_copy(nscore_v, nscore_hbm.at[pl.ds(base, BPW)])

    return scores_kernel(u_idx, v_idx, neg3, u_weight, v_weight)


def _loss_tc_kernel(s_ref, n_ref, o_ref):
    s = s_ref[...]
    ns = n_ref[...]

    def logsig(x):
        return jnp.minimum(x, 0.0) - jnp.log1p(jnp.exp(-jnp.abs(x)))

    total = jnp.sum(logsig(s) + logsig(-ns))
    o_ref[...] = jnp.full((1, 1), -total / BATCH, jnp.float32)


def kernel(pos, v_neg, u_weight, v_weight):
    u_idx = pos[:, 0].astype(jnp.int32)
    v_idx = pos[:, 1].astype(jnp.int32)
    neg3 = (
        v_neg.astype(jnp.int32)
        .reshape(NW, BPW, N_NEG)
        .transpose(0, 2, 1)
    )
    score, nscore = _sc_scores(u_idx, v_idx, neg3, u_weight, v_weight)

    out = pl.pallas_call(
        _loss_tc_kernel,
        out_shape=jax.ShapeDtypeStruct((1, 1), jnp.float32),
    )(score.reshape(NW, BPW), nscore.reshape(NW, BPW))
    return out[0, 0]


# speedup vs baseline: 1.4744x; 1.0359x over previous
import functools

import jax
import jax.numpy as jnp
from jax import lax
from jax.experimental import pallas as pl
from jax.experimental.pallas import tpu as pltpu
from jax.experimental.pallas import tpu_sc as plsc

VOCAB = 100000
DIM = 128
BATCH = 4096
N_NEG = 20
LANES = 16
NC = 2
NS = 16
NW = NC * NS
BPW = BATCH // NW
CHUNKS = DIM // LANES
GROUPS = BPW // LANES
NSEM = 4


def _sc_scores(u_idx, v_idx, neg3, u_weight, v_weight):
    mesh = plsc.VectorSubcoreMesh(core_axis_name="c", subcore_axis_name="s")

    @functools.partial(
        pl.kernel,
        out_type=(
            jax.ShapeDtypeStruct((BATCH,), jnp.float32),
            jax.ShapeDtypeStruct((BATCH,), jnp.float32),
        ),
        mesh=mesh,
        compiler_params=pltpu.CompilerParams(needs_layout_passes=False),
        scratch_types=[
            pltpu.VMEM((BPW,), jnp.int32),
            pltpu.VMEM((BPW,), jnp.int32),
            pltpu.VMEM((N_NEG, BPW), jnp.int32),
            pltpu.VMEM((BPW, DIM), jnp.float32),
            pltpu.VMEM((BPW, DIM), jnp.float32),
            pltpu.VMEM((BPW, DIM), jnp.float32),
            pltpu.VMEM((BPW,), jnp.float32),
            pltpu.VMEM((BPW,), jnp.float32),
            pltpu.SemaphoreType.DMA,
            pltpu.SemaphoreType.DMA,
            pltpu.SemaphoreType.DMA,
            pltpu.SemaphoreType.DMA,
            pltpu.SemaphoreType.DMA,
            pltpu.SemaphoreType.DMA,
            pltpu.SemaphoreType.DMA,
        ],
    )
    def scores_kernel(u_idx_hbm, v_idx_hbm, neg3_hbm, u_w, v_w,
                      score_hbm, nscore_hbm,
                      uidx_v, vidx_v, nidx_v, u_rows, v_rows, negsum,
                      score_v, nscore_v,
                      semu, semv, semi, as0, as1, as2, as3):
        sub = lax.axis_index("s")
        wid = sub * NC + lax.axis_index("c")
        base = pl.multiple_of(wid * BPW, BPW)
        asems = (as0, as1, as2, as3)

        su = pltpu.async_copy(u_idx_hbm.at[pl.ds(base, BPW)], uidx_v, semu)
        sv = pltpu.async_copy(v_idx_hbm.at[pl.ds(base, BPW)], vidx_v, semv)
        sn = pltpu.async_copy(neg3_hbm.at[wid], nidx_v, semi)

        zeros = jnp.zeros((LANES,), jnp.float32)

        def zero_body(item, carry):
            for c in range(CHUNKS):
                negsum[item, pl.ds(c * LANES, LANES)] = zeros
            return carry

        lax.fori_loop(0, BPW, zero_body, 0)

        su.wait()
        cu = pltpu.async_copy(u_w.at[uidx_v], u_rows, semu)
        sv.wait()
        cv = pltpu.async_copy(v_w.at[vidx_v], v_rows, semv)
        sn.wait()
        pend = []
        for n in range(N_NEG):
            pend.append(
                pltpu.async_copy(
                    v_w.at[nidx_v.at[n]], negsum, asems[n % NSEM],
                    add=True,
                )
            )

        lane_iota = jnp.arange(LANES, dtype=jnp.int32)

        cu.wait()
        cv.wait()

        def pos_body(g, carry):
            sp = jnp.zeros((LANES,), jnp.float32)
            for i in range(LANES):
                item = g * LANES + i
                accp = jnp.zeros((LANES,), jnp.float32)
                for c in range(CHUNKS):
                    sl = pl.ds(c * LANES, LANES)
                    accp = accp + u_rows[item, sl] * v_rows[item, sl]
                sp = jnp.where(lane_iota == i, jnp.sum(accp), sp)
            score_v[pl.ds(g * LANES, LANES)] = sp
            return carry

        lax.fori_loop(0, GROUPS, pos_body, 0)
        pltpu.sync_copy(score_v, score_hbm.at[pl.ds(base, BPW)])

        for c in pend:
            c.wait()

        def neg_body(g, carry):
            sn = jnp.zeros((LANES,), jnp.float32)
            for i in range(LANES):
                item = g * LANES + i
                accn = jnp.zeros((LANES,), jnp.float32)
                for c in range(CHUNKS):
                    sl = pl.ds(c * LANES, LANES)
                    accn = accn + u_rows[item, sl] * negsum[item, sl]
                sn = jnp.where(lane_iota == i, jnp.sum(accn), sn)
            nscore_v[pl.ds(g * LANES, LANES)] = sn
            return carry

        lax.fori_loop(0, GROUPS, neg_body, 0)
        pltpu.sync_copy(nscore_v, nscore_hbm.at[pl.ds(base, BPW)])

    return scores_kernel(u_idx, v_idx, neg3, u_weight, v_weight)


def _loss_tc_kernel(s_ref, n_ref, o_ref):
    s = s_ref[...]
    ns = n_ref[...]

    def logsig(x):
        return jnp.minimum(x, 0.0) - jnp.log1p(jnp.exp(-jnp.abs(x)))

    total = jnp.sum(logsig(s) + logsig(-ns))
    o_ref[...] = jnp.full((1, 1), -total / BATCH, jnp.float32)


def kernel(pos, v_neg, u_weight, v_weight):
    u_idx = pos[:, 0].astype(jnp.int32)
    v_idx = pos[:, 1].astype(jnp.int32)
    neg3 = (
        v_neg.astype(jnp.int32)
        .reshape(NW, BPW, N_NEG)
        .transpose(0, 2, 1)
    )
    score, nscore = _sc_scores(u_idx, v_idx, neg3, u_weight, v_weight)

    out = pl.pallas_call(
        _loss_tc_kernel,
        out_shape=jax.ShapeDtypeStruct((1, 1), jnp.float32),
    )(score.reshape(NW, BPW), nscore.reshape(NW, BPW))
    return out[0, 0]
